# baseline (speedup 1.0000x reference)
import functools

import jax
import jax.numpy as jnp
from jax import lax
from jax.experimental import pallas as pl
from jax.experimental.pallas import tpu as pltpu
from jax.experimental.pallas import tpu_sc as plsc

_B = 4
_S = 2048
_D = 768
_CHUNK = 256

_NC = 2
_NS = 16
_L = 16
_NW = _NC * _NS
_RPW = (_B * _S) // _NW
_RCH = 32
_NCHK = _RPW // _RCH

_sc_mesh = plsc.VectorSubcoreMesh(core_axis_name="c", subcore_axis_name="s")


@functools.partial(
    pl.kernel,
    out_type=jax.ShapeDtypeStruct((_NW, _D), jnp.float32),
    mesh=_sc_mesh,
    scratch_types=[
        pltpu.VMEM((_RCH, _D), jnp.float32),
        pltpu.VMEM((_RCH, _D), jnp.float32),
        pltpu.VMEM((_D,), jnp.float32),
        pltpu.SemaphoreType.DMA,
        pltpu.SemaphoreType.DMA,
    ],
)
def _sc_partial_sums(q_hbm, out_hbm, buf0, buf1, acc, sem0, sem1):
    wid = lax.axis_index("s") * _NC + lax.axis_index("c")
    base = wid * _RPW
    for j in range(_D // _L):
        acc[pl.ds(j * _L, _L)] = jnp.zeros((_L,), jnp.float32)

    bufs = (buf0, buf1)
    sems = (sem0, sem1)
    dmas = [None] * _NCHK
    dmas[0] = pltpu.async_copy(
        q_hbm.at[pl.ds(base, _RCH)], bufs[0], sems[0])
    for c in range(_NCHK):
        if c + 1 < _NCHK:
            dmas[c + 1] = pltpu.async_copy(
                q_hbm.at[pl.ds(base + (c + 1) * _RCH, _RCH)],
                bufs[(c + 1) % 2], sems[(c + 1) % 2])
        dmas[c].wait()
        cur = bufs[c % 2]

        def _row(r, carry, cur=cur):
            for j in range(_D // _L):
                plsc.addupdate(acc.at[pl.ds(j * _L, _L)],
                               cur[r, pl.ds(j * _L, _L)])
            return carry

        lax.fori_loop(0, _RCH, _row, jnp.int32(0))
    pltpu.sync_copy(acc, out_hbm.at[wid])


def _tc_reduce_body(x_ref, o_ref, accx):
    i = pl.program_id(0)
    n = pl.num_programs(0)

    @pl.when(i == 0)
    def _init():
        accx[...] = jnp.zeros_like(accx)

    accx[...] += jnp.sum(x_ref[...], axis=1)

    @pl.when(i == n - 1)
    def _finish():
        o_ref[...] = accx[...]


def _tc_attend_body(xs_ref, qp_ref, o_ref):
    xp = xs_ref[...] * (1.0 / _S)
    qp = jnp.sum(qp_ref[...], axis=1) * (1.0 / _S)
    attn = jax.lax.dot_general(
        qp, xp, (((1,), (1,)), ((), ())),
        preferred_element_type=jnp.float32)
    attn = jax.nn.softmax(attn, axis=-1)
    ctx = jnp.dot(attn, xp, preferred_element_type=jnp.float32)
    o_ref[...] = ctx[:, None, :]


def kernel(x, query, keys, values, energy_score):
    del keys, values, energy_score
    qpart = _sc_partial_sums(query.reshape(_B * _S, _D))
    xsum = pl.pallas_call(
        _tc_reduce_body,
        grid=(_S // _CHUNK,),
        in_specs=[pl.BlockSpec((_B, _CHUNK, _D), lambda i: (0, i, 0))],
        out_specs=pl.BlockSpec((_B, _D), lambda i: (0, 0)),
        out_shape=jax.ShapeDtypeStruct((_B, _D), jnp.float32),
        scratch_shapes=[pltpu.VMEM((_B, _D), jnp.float32)],
    )(x)
    return pl.pallas_call(
        _tc_attend_body,
        in_specs=[
            pl.BlockSpec((_B, _D), lambda: (0, 0)),
            pl.BlockSpec((_B, _NW // _B, _D), lambda: (0, 0, 0)),
        ],
        out_specs=pl.BlockSpec((_B, 1, _D), lambda: (0, 0, 0)),
        out_shape=jax.ShapeDtypeStruct((_B, 1, _D), jnp.float32),
    )(xsum, qpart.reshape(_B, _NW // _B, _D))


# speedup vs baseline: 1.3807x; 1.3807x over previous
import functools

import jax
import jax.numpy as jnp
from jax import lax
from jax.experimental import pallas as pl
from jax.experimental.pallas import tpu as pltpu
from jax.experimental.pallas import tpu_sc as plsc

_B = 4
_S = 2048
_D = 768
_CHUNK = 256

_NC = 2
_NS = 16
_L = 16
_NW = _NC * _NS
_RPW = (_B * _S) // _NW
_RCH = 32
_NCHK = _RPW // _RCH
_UNROLL = 4

_sc_mesh = plsc.VectorSubcoreMesh(core_axis_name="c", subcore_axis_name="s")


@functools.partial(
    pl.kernel,
    out_type=jax.ShapeDtypeStruct((_NW, _D), jnp.float32),
    mesh=_sc_mesh,
    scratch_types=[
        pltpu.VMEM((_RCH, _D), jnp.float32),
        pltpu.VMEM((_RCH, _D), jnp.float32),
        pltpu.VMEM((_D,), jnp.float32),
        pltpu.SemaphoreType.DMA,
        pltpu.SemaphoreType.DMA,
    ],
)
def _sc_partial_sums(q_hbm, out_hbm, buf0, buf1, acc, sem0, sem1):
    wid = lax.axis_index("s") * _NC + lax.axis_index("c")
    base = wid * _RPW
    nslice = _D // _L

    bufs = (buf0, buf1)
    sems = (sem0, sem1)
    dmas = [None] * _NCHK
    dmas[0] = pltpu.async_copy(
        q_hbm.at[pl.ds(base, _RCH)], bufs[0], sems[0])
    accs = tuple(jnp.zeros((_L,), jnp.float32) for _ in range(nslice))
    for c in range(_NCHK):
        if c + 1 < _NCHK:
            dmas[c + 1] = pltpu.async_copy(
                q_hbm.at[pl.ds(base + (c + 1) * _RCH, _RCH)],
                bufs[(c + 1) % 2], sems[(c + 1) % 2])
        dmas[c].wait()
        cur = bufs[c % 2]

        def _row(r, carry, cur=cur):
            r0 = r * _UNROLL
            for u in range(_UNROLL):
                carry = tuple(
                    carry[j] + cur[r0 + u, pl.ds(j * _L, _L)]
                    for j in range(nslice))
            return carry

        accs = lax.fori_loop(0, _RCH // _UNROLL, _row, accs)
    for j in range(nslice):
        acc[pl.ds(j * _L, _L)] = accs[j]
    pltpu.sync_copy(acc, out_hbm.at[wid])


def _tc_reduce_body(x_ref, o_ref, accx):
    i = pl.program_id(0)
    n = pl.num_programs(0)

    @pl.when(i == 0)
    def _init():
        accx[...] = jnp.zeros_like(accx)

    accx[...] += jnp.sum(x_ref[...], axis=1)

    @pl.when(i == n - 1)
    def _finish():
        o_ref[...] = accx[...]


def _tc_attend_body(xs_ref, qp_ref, o_ref):
    xp = xs_ref[...] * (1.0 / _S)
    qp = jnp.sum(qp_ref[...], axis=1) * (1.0 / _S)
    attn = jax.lax.dot_general(
        qp, xp, (((1,), (1,)), ((), ())),
        preferred_element_type=jnp.float32)
    attn = jax.nn.softmax(attn, axis=-1)
    ctx = jnp.dot(attn, xp, preferred_element_type=jnp.float32)
    o_ref[...] = ctx[:, None, :]


def kernel(x, query, keys, values, energy_score):
    del keys, values, energy_score
    qpart = _sc_partial_sums(query.reshape(_B * _S, _D))
    xsum = pl.pallas_call(
        _tc_reduce_body,
        grid=(_S // _CHUNK,),
        in_specs=[pl.BlockSpec((_B, _CHUNK, _D), lambda i: (0, i, 0))],
        out_specs=pl.BlockSpec((_B, _D), lambda i: (0, 0)),
        out_shape=jax.ShapeDtypeStruct((_B, _D), jnp.float32),
        scratch_shapes=[pltpu.VMEM((_B, _D), jnp.float32)],
    )(x)
    return pl.pallas_call(
        _tc_attend_body,
        in_specs=[
            pl.BlockSpec((_B, _D), lambda: (0, 0)),
            pl.BlockSpec((_B, _NW // _B, _D), lambda: (0, 0, 0)),
        ],
        out_specs=pl.BlockSpec((_B, 1, _D), lambda: (0, 0, 0)),
        out_shape=jax.ShapeDtypeStruct((_B, 1, _D), jnp.float32),
    )(xsum, qpart.reshape(_B, _NW // _B, _D))


# speedup vs baseline: 4.4194x; 3.2008x over previous
import jax
import jax.numpy as jnp
from jax.experimental import pallas as pl
from jax.experimental.pallas import tpu as pltpu

_B = 4
_S = 2048
_D = 768
_CHUNK = 256


def _body(x_ref, q_ref, o_ref, accx, accq):
    i = pl.program_id(0)
    n = pl.num_programs(0)

    @pl.when(i == 0)
    def _init():
        accx[...] = jnp.zeros_like(accx)
        accq[...] = jnp.zeros_like(accq)

    accx[...] += jnp.sum(x_ref[...], axis=1)
    accq[...] += jnp.sum(q_ref[...], axis=1)

    @pl.when(i == n - 1)
    def _finish():
        xp = accx[...] * (1.0 / _S)
        qp = accq[...] * (1.0 / _S)
        attn = jax.lax.dot_general(
            qp, xp, (((1,), (1,)), ((), ())),
            preferred_element_type=jnp.float32)
        attn = jax.nn.softmax(attn, axis=-1)
        ctx = jnp.dot(attn, xp, preferred_element_type=jnp.float32)
        o_ref[...] = ctx[:, None, :]


def kernel(x, query, keys, values, energy_score):
    del keys, values, energy_score
    return pl.pallas_call(
        _body,
        grid=(_S // _CHUNK,),
        in_specs=[
            pl.BlockSpec((_B, _CHUNK, _D), lambda i: (0, i, 0)),
            pl.BlockSpec((_B, _CHUNK, _D), lambda i: (0, i, 0)),
        ],
        out_specs=pl.BlockSpec((_B, 1, _D), lambda i: (0, 0, 0)),
        out_shape=jax.ShapeDtypeStruct((_B, 1, _D), jnp.float32),
        scratch_shapes=[
            pltpu.VMEM((_B, _D), jnp.float32),
            pltpu.VMEM((_B, _D), jnp.float32),
        ],
    )(x, query)
